# baseline (speedup 1.0000x reference)
import jax
import jax.numpy as jnp
from jax import lax
from jax.experimental import pallas as pl
from jax.experimental.pallas import tpu as pltpu
from jax.experimental.pallas import tpu_sc as plsc

VOCAB_PAD = 1024
EMB = 64
SCALE = 8.0
NC = 2
NS = 16
NW = NC * NS
B_TOTAL = 4096 * 200
B_PER_W = B_TOTAL // NW
CHUNK = 640
N_CHUNKS = B_PER_W // CHUNK
ROWS_PER_TILE = VOCAB_PAD // NS


def _body(x_hbm, tab_hbm, out_hbm, shared, idxbuf, rows0, rows1,
          gsem0, gsem1, osem0, osem1):
    s = lax.axis_index("s")
    wid = s * NC + lax.axis_index("c")
    rows = (rows0, rows1)
    gsem = (gsem0, gsem1)
    osem = (osem0, osem1)

    tstage = rows0.at[pl.ds(0, ROWS_PER_TILE)]
    pltpu.sync_copy(tab_hbm.at[pl.ds(s * ROWS_PER_TILE, ROWS_PER_TILE)], tstage)

    def scale_row(r, carry):
        for j in range(EMB // 16):
            rows0[r, pl.ds(j * 16, 16)] = rows0[r, pl.ds(j * 16, 16)] * SCALE
        return carry

    lax.fori_loop(0, ROWS_PER_TILE, scale_row, 0)
    pltpu.sync_copy(tstage, shared.at[pl.ds(s * ROWS_PER_TILE, ROWS_PER_TILE)])

    pltpu.sync_copy(x_hbm.at[pl.ds(wid * B_PER_W, B_PER_W)], idxbuf)
    plsc.subcore_barrier()

    def issue_gather(g, b):
        pltpu.async_copy(
            shared.at[idxbuf.at[pl.ds(g * CHUNK, CHUNK)]],
            rows[b], gsem[b])

    def wait_chunk(sem, b):
        pltpu.make_async_copy(out_hbm.at[0], rows[b], sem).wait()

    issue_gather(0, 0)
    cbase = wid * N_CHUNKS

    def pair(gg, carry):
        for b in range(2):
            bp = 1 - b
            g = gg * 2 + b
            wait_chunk(gsem[b], b)

            @pl.when(g + 1 < N_CHUNKS)
            def _():
                @pl.when(g >= 1)
                def _():
                    wait_chunk(osem[bp], bp)
                issue_gather(g + 1, bp)

            pltpu.async_copy(rows[b], out_hbm.at[cbase + g], osem[b])
        return carry

    lax.fori_loop(0, N_CHUNKS // 2, pair, 0)
    wait_chunk(osem[0], 0)
    wait_chunk(osem[1], 1)


_sc_call = pl.kernel(
    _body,
    out_type=jax.ShapeDtypeStruct((NW * N_CHUNKS, CHUNK, EMB), jnp.float32),
    mesh=plsc.VectorSubcoreMesh(
        core_axis_name="c", subcore_axis_name="s", num_cores=NC, num_subcores=NS
    ),
    scratch_types=[
        pltpu.VMEM_SHARED((VOCAB_PAD, EMB), jnp.float32),
        pltpu.VMEM((B_PER_W,), jnp.int32),
        pltpu.VMEM((CHUNK, EMB), jnp.float32),
        pltpu.VMEM((CHUNK, EMB), jnp.float32),
        pltpu.SemaphoreType.DMA,
        pltpu.SemaphoreType.DMA,
        pltpu.SemaphoreType.DMA,
        pltpu.SemaphoreType.DMA,
    ],
    compiler_params=pltpu.CompilerParams(use_tc_tiling_on_sc=False),
)


def kernel(x, table):
    tab = jnp.pad(table, ((0, VOCAB_PAD - table.shape[0]), (0, 0)))
    out = _sc_call(x.reshape(-1), tab)
    return out.reshape(x.shape[0], x.shape[1], EMB)
